# Initial kernel scaffold; baseline (speedup 1.0000x reference)
import functools

import jax
import jax.numpy as jnp
from jax import lax
from jax.experimental import pallas as pl
from jax.experimental.pallas import tpu as pltpu
from jax.experimental.pallas import tpu_sc as plsc

_BLK = 128
_CH_BLK = 4


@functools.lru_cache(maxsize=None)
def _make_gather(rows_total: int, table_rows: int, d: int):
    info = plsc.get_sparse_core_info()
    nw = info.num_cores * info.num_subcores
    n_blocks = rows_total // _BLK
    blocks_per_w = n_blocks // nw
    n_chunks = blocks_per_w // _CH_BLK
    ch_rows = _CH_BLK * _BLK
    assert n_blocks % nw == 0 and blocks_per_w % _CH_BLK == 0

    mesh = plsc.VectorSubcoreMesh(core_axis_name="c", subcore_axis_name="s")

    @functools.partial(
        pl.kernel,
        mesh=mesh,
        out_type=jax.ShapeDtypeStruct((rows_total, d), jnp.float32),
        scratch_types=[
            pltpu.VMEM((_CH_BLK, _BLK), jnp.int32),
            pltpu.VMEM((_CH_BLK, _BLK), jnp.int32),
            pltpu.VMEM((ch_rows, d), jnp.float32),
            pltpu.VMEM((ch_rows, d), jnp.float32),
            pltpu.SemaphoreType.DMA,
            pltpu.SemaphoreType.DMA,
            pltpu.SemaphoreType.DMA,
            pltpu.SemaphoreType.DMA,
        ],
    )
    def gather_kernel(table_hbm, idx_hbm, out_hbm,
                      idx0, idx1, rows0, rows1, g0, g1, w0, w1):
        wid = lax.axis_index("s") * info.num_cores + lax.axis_index("c")
        wblk0 = wid * blocks_per_w

        idx_bufs = (idx0, idx1)
        row_bufs = (rows0, rows1)
        gsems = (g0, g1)
        wsems = (w0, w1)

        gather_handles = [[], []]
        write_handles = [None, None]

        def start_gather(ci, b):
            blk0 = wblk0 + ci * _CH_BLK
            pltpu.sync_copy(idx_hbm.at[pl.ds(blk0, _CH_BLK)], idx_bufs[b])
            hs = []
            for j in range(_CH_BLK):
                hs.append(pltpu.async_copy(
                    table_hbm.at[idx_bufs[b].at[j]],
                    row_bufs[b].at[pl.ds(j * _BLK, _BLK)],
                    gsems[b]))
            gather_handles[b] = hs

        start_gather(0, 0)
        for ci in range(n_chunks):
            b = ci % 2
            if ci + 1 < n_chunks:
                nb = 1 - b
                if write_handles[nb] is not None:
                    write_handles[nb].wait()
                    write_handles[nb] = None
                start_gather(ci + 1, nb)
            for h in gather_handles[b]:
                h.wait()
            row0 = (wblk0 + ci * _CH_BLK) * _BLK
            write_handles[b] = pltpu.async_copy(
                row_bufs[b], out_hbm.at[pl.ds(row0, ch_rows)], wsems[b])
        for b in range(2):
            if write_handles[b] is not None:
                write_handles[b].wait()

    return gather_kernel


def kernel(x, refer_idx):
    b, n, d = x.shape
    k = refer_idx.shape[2]
    rows_total = b * n * k
    table = x.reshape(b * n, d)
    base = (jnp.arange(b, dtype=jnp.int32) * n).reshape(b, 1, 1)
    idx = (refer_idx.astype(jnp.int32) + base).reshape(rows_total // _BLK, _BLK)
    out = _make_gather(rows_total, b * n, d)(table, idx)
    return out.reshape(b, n, k * d)


# speedup vs baseline: 6.6608x; 6.6608x over previous
import functools

import jax
import jax.numpy as jnp
from jax import lax
from jax.experimental import pallas as pl
from jax.experimental.pallas import tpu as pltpu
from jax.experimental.pallas import tpu_sc as plsc

_BLK = 128
_CH_BLK = 4


@functools.lru_cache(maxsize=None)
def _make_gather(rows_total: int, table_rows: int, d: int):
    info = plsc.get_sparse_core_info()
    nw = info.num_cores * info.num_subcores
    n_blocks = rows_total // _BLK
    blocks_per_w = n_blocks // nw
    n_chunks = blocks_per_w // _CH_BLK
    ch_rows = _CH_BLK * _BLK
    assert n_blocks % nw == 0 and blocks_per_w % _CH_BLK == 0

    mesh = plsc.VectorSubcoreMesh(core_axis_name="c", subcore_axis_name="s")

    @functools.partial(
        pl.kernel,
        mesh=mesh,
        compiler_params=pltpu.CompilerParams(use_tc_tiling_on_sc=False),
        out_type=jax.ShapeDtypeStruct((rows_total, d), jnp.float32),
        scratch_types=[
            pltpu.VMEM((_CH_BLK, _BLK), jnp.int32),
            pltpu.VMEM((_CH_BLK, _BLK), jnp.int32),
            pltpu.VMEM((ch_rows, d), jnp.float32),
            pltpu.VMEM((ch_rows, d), jnp.float32),
            pltpu.SemaphoreType.DMA,
            pltpu.SemaphoreType.DMA,
            pltpu.SemaphoreType.DMA,
            pltpu.SemaphoreType.DMA,
        ],
    )
    def gather_kernel(table_hbm, idx_hbm, out_hbm,
                      idx0, idx1, rows0, rows1, g0, g1, w0, w1):
        wid = lax.axis_index("s") * info.num_cores + lax.axis_index("c")
        wblk0 = wid * blocks_per_w

        idx_bufs = (idx0, idx1)
        row_bufs = (rows0, rows1)
        gsems = (g0, g1)
        wsems = (w0, w1)

        gather_handles = [[], []]
        write_handles = [None, None]

        def start_gather(ci, b):
            blk0 = wblk0 + ci * _CH_BLK
            pltpu.sync_copy(idx_hbm.at[pl.ds(blk0, _CH_BLK)], idx_bufs[b])
            hs = []
            for j in range(_CH_BLK):
                hs.append(pltpu.async_copy(
                    table_hbm.at[idx_bufs[b].at[j]],
                    row_bufs[b].at[pl.ds(j * _BLK, _BLK)],
                    gsems[b]))
            gather_handles[b] = hs

        start_gather(0, 0)
        for ci in range(n_chunks):
            b = ci % 2
            if ci + 1 < n_chunks:
                nb = 1 - b
                if write_handles[nb] is not None:
                    write_handles[nb].wait()
                    write_handles[nb] = None
                start_gather(ci + 1, nb)
            for h in gather_handles[b]:
                h.wait()
            row0 = (wblk0 + ci * _CH_BLK) * _BLK
            write_handles[b] = pltpu.async_copy(
                row_bufs[b], out_hbm.at[pl.ds(row0, ch_rows)], wsems[b])
        for b in range(2):
            if write_handles[b] is not None:
                write_handles[b].wait()

    return gather_kernel


def kernel(x, refer_idx):
    b, n, d = x.shape
    k = refer_idx.shape[2]
    rows_total = b * n * k
    table = x.reshape(b * n, d)
    base = (jnp.arange(b, dtype=jnp.int32) * n).reshape(b, 1, 1)
    idx = (refer_idx.astype(jnp.int32) + base).reshape(rows_total // _BLK, _BLK)
    out = _make_gather(rows_total, b * n, d)(table, idx)
    return out.reshape(b, n, k * d)


# speedup vs baseline: 6.7821x; 1.0182x over previous
import functools

import jax
import jax.numpy as jnp
from jax import lax
from jax.experimental import pallas as pl
from jax.experimental.pallas import tpu as pltpu
from jax.experimental.pallas import tpu_sc as plsc

_BLK = 128
_CH_BLK = 2


@functools.lru_cache(maxsize=None)
def _make_gather(rows_total: int, table_rows: int, d: int):
    info = plsc.get_sparse_core_info()
    nw = info.num_cores * info.num_subcores
    n_blocks = rows_total // _BLK
    blocks_per_w = n_blocks // nw
    n_chunks = blocks_per_w // _CH_BLK
    ch_rows = _CH_BLK * _BLK
    assert n_blocks % nw == 0 and blocks_per_w % _CH_BLK == 0

    mesh = plsc.VectorSubcoreMesh(core_axis_name="c", subcore_axis_name="s")
    nbuf = 4
    ahead = 2

    @functools.partial(
        pl.kernel,
        mesh=mesh,
        compiler_params=pltpu.CompilerParams(use_tc_tiling_on_sc=False),
        out_type=jax.ShapeDtypeStruct((rows_total, d), jnp.float32),
        scratch_types=[
            pltpu.VMEM((blocks_per_w, _BLK), jnp.int32),
            [pltpu.VMEM((ch_rows, d), jnp.float32)] * nbuf,
            [pltpu.SemaphoreType.DMA] * nbuf,
            [pltpu.SemaphoreType.DMA] * nbuf,
        ],
    )
    def gather_kernel(table_hbm, idx_hbm, out_hbm,
                      idx_v, row_bufs, gsems, wsems):
        wid = lax.axis_index("s") * info.num_cores + lax.axis_index("c")
        wblk0 = wid * blocks_per_w

        pltpu.sync_copy(idx_hbm.at[pl.ds(wblk0, blocks_per_w)], idx_v)

        gather_handles = [[] for _ in range(nbuf)]
        write_handles = [None] * nbuf

        def start_gather(ci):
            b = ci % nbuf
            hs = []
            for j in range(_CH_BLK):
                hs.append(pltpu.async_copy(
                    table_hbm.at[idx_v.at[ci * _CH_BLK + j]],
                    row_bufs[b].at[pl.ds(j * _BLK, _BLK)],
                    gsems[b]))
            gather_handles[b] = hs

        for ci in range(min(ahead, n_chunks)):
            start_gather(ci)
        for ci in range(n_chunks):
            b = ci % nbuf
            for h in gather_handles[b]:
                h.wait()
            row0 = (wblk0 + ci * _CH_BLK) * _BLK
            write_handles[b] = pltpu.async_copy(
                row_bufs[b], out_hbm.at[pl.ds(row0, ch_rows)], wsems[b])
            nxt = ci + ahead
            if nxt < n_chunks:
                nb = nxt % nbuf
                if write_handles[nb] is not None:
                    write_handles[nb].wait()
                    write_handles[nb] = None
                start_gather(nxt)
        for b in range(nbuf):
            if write_handles[b] is not None:
                write_handles[b].wait()

    return gather_kernel


def kernel(x, refer_idx):
    b, n, d = x.shape
    k = refer_idx.shape[2]
    rows_total = b * n * k
    table = x.reshape(b * n, d)
    base = (jnp.arange(b, dtype=jnp.int32) * n).reshape(b, 1, 1)
    idx = (refer_idx.astype(jnp.int32) + base).reshape(rows_total // _BLK, _BLK)
    out = _make_gather(rows_total, b * n, d)(table, idx)
    return out.reshape(b, n, k * d)


# speedup vs baseline: 6.8076x; 1.0038x over previous
import functools

import jax
import jax.numpy as jnp
from jax import lax
from jax.experimental import pallas as pl
from jax.experimental.pallas import tpu as pltpu
from jax.experimental.pallas import tpu_sc as plsc

_BLK = 128
_CH_BLK = 2


@functools.lru_cache(maxsize=None)
def _make_gather(n_batch: int, n_points: int, d: int, k: int):
    rows_total = n_batch * n_points * k
    info = plsc.get_sparse_core_info()
    nw = info.num_cores * info.num_subcores
    n_blocks = rows_total // _BLK
    blocks_per_w = n_blocks // nw
    n_chunks = blocks_per_w // _CH_BLK
    ch_rows = _CH_BLK * _BLK
    assert n_blocks % nw == 0 and blocks_per_w % _CH_BLK == 0
    assert nw % n_batch == 0

    mesh = plsc.VectorSubcoreMesh(core_axis_name="c", subcore_axis_name="s")
    nbuf = 4
    ahead = 2

    @functools.partial(
        pl.kernel,
        mesh=mesh,
        compiler_params=pltpu.CompilerParams(use_tc_tiling_on_sc=False),
        out_type=jax.ShapeDtypeStruct((rows_total, d), jnp.float32),
        scratch_types=[
            pltpu.VMEM((blocks_per_w, _BLK), jnp.int32),
            [pltpu.VMEM((ch_rows, d), jnp.float32)] * nbuf,
            [pltpu.SemaphoreType.DMA] * nbuf,
            [pltpu.SemaphoreType.DMA] * nbuf,
        ],
    )
    def gather_kernel(x_hbm, idx_hbm, out_hbm,
                      idx_v, row_bufs, gsems, wsems):
        wid = lax.axis_index("s") * info.num_cores + lax.axis_index("c")
        wblk0 = wid * blocks_per_w
        bidx = wid // (nw // n_batch)
        table = x_hbm.at[bidx]

        pltpu.sync_copy(idx_hbm.at[pl.ds(wblk0, blocks_per_w)], idx_v)

        gather_handles = [[] for _ in range(nbuf)]
        write_handles = [None] * nbuf

        def start_gather(ci):
            b = ci % nbuf
            hs = []
            for j in range(_CH_BLK):
                hs.append(pltpu.async_copy(
                    table.at[idx_v.at[ci * _CH_BLK + j]],
                    row_bufs[b].at[pl.ds(j * _BLK, _BLK)],
                    gsems[b]))
            gather_handles[b] = hs

        for ci in range(min(ahead, n_chunks)):
            start_gather(ci)
        for ci in range(n_chunks):
            b = ci % nbuf
            for h in gather_handles[b]:
                h.wait()
            row0 = (wblk0 + ci * _CH_BLK) * _BLK
            write_handles[b] = pltpu.async_copy(
                row_bufs[b], out_hbm.at[pl.ds(row0, ch_rows)], wsems[b])
            nxt = ci + ahead
            if nxt < n_chunks:
                nb = nxt % nbuf
                if write_handles[nb] is not None:
                    write_handles[nb].wait()
                    write_handles[nb] = None
                start_gather(nxt)
        for b in range(nbuf):
            if write_handles[b] is not None:
                write_handles[b].wait()

    return gather_kernel


def kernel(x, refer_idx):
    b, n, d = x.shape
    k = refer_idx.shape[2]
    idx = refer_idx.astype(jnp.int32).reshape((b * n * k) // _BLK, _BLK)
    out = _make_gather(b, n, d, k)(x, idx)
    return out.reshape(b, n, k * d)


# speedup vs baseline: 6.8874x; 1.0117x over previous
import functools

import jax
import jax.numpy as jnp
from jax import lax
from jax.experimental import pallas as pl
from jax.experimental.pallas import tpu as pltpu
from jax.experimental.pallas import tpu_sc as plsc

_BLK = 128
_CH_BLK = 2
_NBUF = 4
_AHEAD = 2


@functools.lru_cache(maxsize=None)
def _make_gather(n_batch: int, n_points: int, d: int, k: int):
    rows_total = n_batch * n_points * k
    info = plsc.get_sparse_core_info()
    nw = info.num_cores * info.num_subcores
    rows_per_w = rows_total // nw
    n_chunks = rows_per_w // (_CH_BLK * _BLK)
    ch_rows = _CH_BLK * _BLK
    pts_per_ch = ch_rows // k
    lanes = info.num_lanes
    assert rows_total % (nw * ch_rows) == 0 and n_chunks % _NBUF == 0
    assert nw % n_batch == 0 and k == lanes

    mesh = plsc.VectorSubcoreMesh(core_axis_name="c", subcore_axis_name="s")

    @functools.partial(
        pl.kernel,
        mesh=mesh,
        compiler_params=pltpu.CompilerParams(use_tc_tiling_on_sc=False),
        out_type=jax.ShapeDtypeStruct((rows_total, d), jnp.float32),
        scratch_types=[
            pltpu.VMEM((rows_per_w // k, k), jnp.int32),
            [pltpu.VMEM((_CH_BLK, _BLK), jnp.int32)] * _NBUF,
            [pltpu.VMEM((ch_rows, d), jnp.float32)] * _NBUF,
            [pltpu.SemaphoreType.DMA] * _NBUF,
            [pltpu.SemaphoreType.DMA] * _NBUF,
        ],
    )
    def gather_kernel(x_hbm, idx_hbm, out_hbm,
                      slab, gidx, row_bufs, gsems, wsems):
        wid = lax.axis_index("s") * info.num_cores + lax.axis_index("c")
        bidx = wid // (nw // n_batch)
        p_base = (wid % (nw // n_batch)) * (rows_per_w // k)
        wrow0 = wid * rows_per_w
        table = x_hbm.at[bidx]

        pltpu.sync_copy(idx_hbm.at[bidx, pl.ds(p_base, rows_per_w // k)], slab)

        def repack(ci, b):
            for p in range(pts_per_ch):
                vec = slab[ci * pts_per_ch + p, :]
                gidx[b][p * k // _BLK, pl.ds((p * k) % _BLK, k)] = vec

        def start_gather(ci, b):
            for j in range(_CH_BLK):
                pltpu.async_copy(
                    table.at[gidx[b].at[j]],
                    row_bufs[b].at[pl.ds(j * _BLK, _BLK)],
                    gsems[b])

        def drain_gather(ci, b):
            for j in range(_CH_BLK):
                pltpu.make_async_copy(
                    table.at[gidx[b].at[j]],
                    row_bufs[b].at[pl.ds(j * _BLK, _BLK)],
                    gsems[b]).wait()

        def write_out(ci, b):
            pltpu.async_copy(
                row_bufs[b], out_hbm.at[pl.ds(wrow0 + ci * ch_rows, ch_rows)],
                wsems[b])

        def drain_write(ci, b):
            pltpu.make_async_copy(
                row_bufs[b], out_hbm.at[pl.ds(wrow0 + ci * ch_rows, ch_rows)],
                wsems[b]).wait()

        for ci in range(_AHEAD):
            repack(ci, ci % _NBUF)
            start_gather(ci, ci % _NBUF)

        def group(g, _):
            for b in range(_NBUF):
                ci = g * _NBUF + b
                drain_gather(ci, b)
                write_out(ci, b)
                nb = (b + _AHEAD) % _NBUF

                @pl.when(ci >= _AHEAD)
                def _():
                    drain_write(ci - _AHEAD, nb)

                @pl.when(ci + _AHEAD < n_chunks)
                def _():
                    repack(ci + _AHEAD, nb)
                    start_gather(ci + _AHEAD, nb)
            return ()

        lax.fori_loop(0, n_chunks // _NBUF, group, (), unroll=False)

        for ci in range(n_chunks - _AHEAD, n_chunks):
            drain_write(ci, ci % _NBUF)

    return gather_kernel


def kernel(x, refer_idx):
    b, n, d = x.shape
    k = refer_idx.shape[2]
    out = _make_gather(b, n, d, k)(x, refer_idx.astype(jnp.int32))
    return out.reshape(b, n, k * d)
